# Initial kernel scaffold; baseline (speedup 1.0000x reference)
import jax
import jax.numpy as jnp
from jax.experimental import pallas as pl


def kernel(x, pos, edge_index, edge_attr, batch_indices, masked_token, enc_W_msg, enc_b_msg, enc_W_upd, enc_b_upd, enc_w_coord, dec_W_msg, dec_b_msg, dec_W_upd, dec_b_upd, dec_w_coord, dec_W_out, dec_b_out):
    raise NotImplementedError("write your pallas kernel here")


# speedup vs baseline: 4.0143x; 4.0143x over previous
import functools

import jax
import jax.numpy as jnp
from jax import lax
from jax.experimental import pallas as pl
from jax.experimental.pallas import tpu as pltpu
from jax.experimental.pallas import tpu_sc as plsc

N = 10000
E = 320000
D = 128
P = 3
DE = 16
NUM_MASKED = 5000
NW = 32
EPW = E // NW
EPT = E // 16
CHUNK = 80
GROUPS = CHUNK // 16
MPAD = 5120
MGW = MPAD // NW
NSPL = N // 2
NPAD = NSPL + 8

BN = 2000
BE = 512

_f32 = jnp.float32
_i32 = jnp.int32


def _consts():
    perm = jax.random.permutation(jax.random.key(42), N)
    mask_idx = perm[:NUM_MASKED].astype(_i32)
    vis_idx = perm[NUM_MASKED:]
    visf = jnp.zeros((N, 1), _f32).at[vis_idx, 0].set(1.0)
    pos_m = jax.random.normal(jax.random.key(7), (NUM_MASKED, P), _f32)
    pmf = jnp.zeros((N, P), _f32).at[mask_idx].set(pos_m)
    mask_pad = jnp.concatenate(
        [mask_idx, jnp.zeros((MPAD - NUM_MASKED,), _i32)])
    return mask_idx, visf, pmf, mask_pad


def _tables_body(x_ref, Wsd_ref, bmsg_ref, tS_ref, tD_ref):
    xb = x_ref[...]
    tS_ref[...] = jnp.dot(xb, Wsd_ref[0],
                          preferred_element_type=_f32) + bmsg_ref[...]
    tD_ref[...] = jnp.dot(xb, Wsd_ref[1], preferred_element_type=_f32)


def _tables_call(xn, Wsd, bmsg):
    return pl.pallas_call(
        _tables_body,
        grid=(N // BN,),
        in_specs=[
            pl.BlockSpec((BN, D), lambda i: (i, 0)),
            pl.BlockSpec((2, D, D), lambda i: (0, 0, 0)),
            pl.BlockSpec((1, D), lambda i: (0, 0)),
        ],
        out_specs=[
            pl.BlockSpec((BN, D), lambda i: (i, 0)),
            pl.BlockSpec((BN, D), lambda i: (i, 0)),
        ],
        out_shape=[
            jax.ShapeDtypeStruct((N, D), _f32),
            jax.ShapeDtypeStruct((N, D), _f32),
        ],
    )(xn, Wsd, bmsg)


_MESH = plsc.VectorSubcoreMesh(core_axis_name="c", subcore_axis_name="s")
_SC_PARAMS = pltpu.CompilerParams(needs_layout_passes=False)


@functools.partial(
    pl.kernel,
    out_type=(jax.ShapeDtypeStruct((E, D), _f32),
              jax.ShapeDtypeStruct((E, D), _f32),
              jax.ShapeDtypeStruct((E,), _f32),
              jax.ShapeDtypeStruct((E,), _f32),
              jax.ShapeDtypeStruct((E,), _f32),
              jax.ShapeDtypeStruct((E,), _f32),
              jax.ShapeDtypeStruct((E,), _f32)),
    mesh=_MESH,
    compiler_params=_SC_PARAMS,
    scratch_types=[
        pltpu.VMEM((4 * N,), _f32),
        pltpu.VMEM((CHUNK,), _i32),
        pltpu.VMEM((CHUNK,), _i32),
        pltpu.VMEM((CHUNK, D), _f32),
        pltpu.VMEM((CHUNK,), _f32),
        pltpu.VMEM((CHUNK,), _f32),
        pltpu.VMEM((CHUNK,), _f32),
        pltpu.VMEM((CHUNK,), _f32),
        pltpu.VMEM((CHUNK,), _f32),
        pltpu.SemaphoreType.DMA,
    ],
)
def _sc_gather(tS, tD, posvis, src, dst, gS, gD, d2o, keepo, rxo, ryo, rzo,
               pv, idx_s, idx_d, rows, d2b, kb, rxb, ryb, rzb, sem):
    wid = lax.axis_index("s") * 2 + lax.axis_index("c")
    base0 = wid * EPW
    pltpu.sync_copy(posvis, pv)

    def step(i, _):
        base = base0 + i * CHUNK
        pltpu.sync_copy(src.at[pl.ds(base, CHUNK)], idx_s)
        pltpu.sync_copy(dst.at[pl.ds(base, CHUNK)], idx_d)
        cp = pltpu.async_copy(tS.at[idx_s], rows, sem)

        def group(g, _):
            sl = pl.ds(g * 16, 16)
            s16 = idx_s[sl]
            d16 = idx_d[sl]
            rx = plsc.load_gather(pv, [s16]) - plsc.load_gather(pv, [d16])
            ry = (plsc.load_gather(pv, [s16 + N])
                  - plsc.load_gather(pv, [d16 + N]))
            rz = (plsc.load_gather(pv, [s16 + 2 * N])
                  - plsc.load_gather(pv, [d16 + 2 * N]))
            kv = (plsc.load_gather(pv, [s16 + 3 * N])
                  * plsc.load_gather(pv, [d16 + 3 * N]))
            d2b[sl] = rx * rx + ry * ry + rz * rz
            kb[sl] = kv
            rxb[sl] = rx
            ryb[sl] = ry
            rzb[sl] = rz
            return 0

        lax.fori_loop(0, GROUPS, group, 0)
        pltpu.sync_copy(d2b, d2o.at[pl.ds(base, CHUNK)])
        pltpu.sync_copy(kb, keepo.at[pl.ds(base, CHUNK)])
        pltpu.sync_copy(rxb, rxo.at[pl.ds(base, CHUNK)])
        pltpu.sync_copy(ryb, ryo.at[pl.ds(base, CHUNK)])
        pltpu.sync_copy(rzb, rzo.at[pl.ds(base, CHUNK)])
        cp.wait()
        pltpu.sync_copy(rows, gS.at[pl.ds(base, CHUNK)])
        cp2 = pltpu.async_copy(tD.at[idx_d], rows, sem)
        cp2.wait()
        pltpu.sync_copy(rows, gD.at[pl.ds(base, CHUNK)])
        return 0

    lax.fori_loop(0, EPW // CHUNK, step, 0)


def _edge_body(gS_ref, gD_ref, eax_ref, We3_ref, wc_ref, mk_ref, cf_ref):
    gs = gS_ref[...]
    gd = gD_ref[...]
    eax = eax_ref[...]
    keep = eax[:, DE + 1:DE + 2]
    m = jnp.maximum(
        gs + gd + jnp.dot(eax, We3_ref[...], preferred_element_type=_f32),
        0.0)
    mk = m * keep
    coefk = jnp.tanh(jnp.dot(mk, wc_ref[...],
                             preferred_element_type=_f32)) * keep
    mk_ref[...] = mk
    cf_ref[...] = coefk.reshape((coefk.shape[0],))


def _edge_call(gS, gD, eax, We3, wc):
    return pl.pallas_call(
        _edge_body,
        grid=(E // BE,),
        in_specs=[
            pl.BlockSpec((BE, D), lambda i: (i, 0)),
            pl.BlockSpec((BE, D), lambda i: (i, 0)),
            pl.BlockSpec((BE, DE + 2), lambda i: (i, 0)),
            pl.BlockSpec((DE + 2, D), lambda i: (0, 0)),
            pl.BlockSpec((D, 1), lambda i: (0, 0)),
        ],
        out_specs=[
            pl.BlockSpec((BE, D), lambda i: (i, 0)),
            pl.BlockSpec((BE,), lambda i: (i,)),
        ],
        out_shape=[
            jax.ShapeDtypeStruct((E, D), _f32),
            jax.ShapeDtypeStruct((E,), _f32),
        ],
    )(gS, gD, eax, We3, wc)


N4 = 4 * N


@functools.partial(
    pl.kernel,
    out_type=(jax.ShapeDtypeStruct((2, NPAD, D), _f32),
              jax.ShapeDtypeStruct((1, N4), _f32)),
    mesh=_MESH,
    compiler_params=_SC_PARAMS,
    scratch_types=[
        pltpu.VMEM((CHUNK,), _i32),
        pltpu.VMEM((CHUNK,), _i32),
        pltpu.VMEM((CHUNK, D), _f32),
        pltpu.VMEM((CHUNK,), _f32),
        pltpu.VMEM((CHUNK,), _f32),
        pltpu.VMEM((CHUNK,), _f32),
        pltpu.VMEM((CHUNK,), _f32),
        pltpu.VMEM((CHUNK,), _f32),
        pltpu.VMEM((N4,), _f32),
        pltpu.VMEM_SHARED((NPAD, D), _f32),
        pltpu.VMEM_SHARED((N4,), _f32),
        pltpu.SemaphoreType.DMA,
    ],
)
def _sc_scatter(vals, dstI, coef, keepf, relx, rely, relz, zerosND, zeros4N,
                iota4n, parts, part4, idx_v, idx_l, buf_v, cfb, kb, rxb, ryb,
                rzb, acc4, acc, accn, sem):
    c = lax.axis_index("c")
    s = lax.axis_index("s")

    def zstep(j, _):
        cid = s + j * 16

        @pl.when(cid < NSPL // 200)
        def _():
            pltpu.sync_copy(zerosND.at[pl.ds(cid * 200, 200)],
                            acc.at[pl.ds(cid * 200, 200)])

        return 0

    lax.fori_loop(0, 2, zstep, 0)

    @pl.when(s == 1)
    def _():
        pltpu.sync_copy(zeros4N, accn)

    @pl.when(c == 0)
    def _():
        pltpu.sync_copy(zeros4N, acc4)

    plsc.subcore_barrier()
    base0 = s * EPT
    lo = c * NSPL

    def step(i, _):
        base = base0 + i * CHUNK
        pltpu.sync_copy(dstI.at[pl.ds(base, CHUNK)], idx_v)
        cp = pltpu.async_copy(vals.at[pl.ds(base, CHUNK)], buf_v, sem)
        pltpu.sync_copy(coef.at[pl.ds(base, CHUNK)], cfb)
        pltpu.sync_copy(keepf.at[pl.ds(base, CHUNK)], kb)
        pltpu.sync_copy(relx.at[pl.ds(base, CHUNK)], rxb)
        pltpu.sync_copy(rely.at[pl.ds(base, CHUNK)], ryb)
        pltpu.sync_copy(relz.at[pl.ds(base, CHUNK)], rzb)

        def group(g, _):
            sl = pl.ds(g * 16, 16)
            d16 = idx_v[sl]
            loc = d16 - lo
            idx_l[sl] = jnp.where((loc >= 0) & (loc < NSPL), loc, NSPL)

            @pl.when(c == 0)
            def _():
                cf = cfb[sl]
                plsc.addupdate_scatter(acc4, [d16], rxb[sl] * cf)
                plsc.addupdate_scatter(acc4, [d16 + N], ryb[sl] * cf)
                plsc.addupdate_scatter(acc4, [d16 + 2 * N], rzb[sl] * cf)
                plsc.addupdate_scatter(acc4, [d16 + 3 * N], kb[sl])

            return 0

        lax.fori_loop(0, GROUPS, group, 0)
        cp.wait()
        pltpu.sync_copy(buf_v, acc.at[idx_l], add=True)
        return 0

    lax.fori_loop(0, EPT // CHUNK, step, 0)

    @pl.when(c == 0)
    def _():
        def nstep(k, _):
            pltpu.sync_copy(iota4n.at[pl.ds(k * CHUNK, CHUNK)], idx_v)
            pltpu.sync_copy(acc4.at[pl.ds(k * CHUNK, CHUNK)],
                            accn.at[idx_v], add=True)
            return 0

        lax.fori_loop(0, N4 // CHUNK, nstep, 0)

    plsc.subcore_barrier()

    def wstep(j, _):
        cid = s + j * 16

        @pl.when(cid < NSPL // 200)
        def _():
            pltpu.sync_copy(acc.at[pl.ds(cid * 200, 200)],
                            parts.at[c, pl.ds(cid * 200, 200)])

        return 0

    lax.fori_loop(0, 2, wstep, 0)

    @pl.when((s == 1) & (c == 0))
    def _():
        pltpu.sync_copy(accn, part4.at[0])


def _upd_body(parts_ref, p4_ref, x_ref, pos_ref, vis_ref, Wu_ref, bu_ref,
              mt_ref, pmf_ref, Wo_ref, bo_ref, z_ref, posc_ref, pr_ref, *,
              decoder):
    acc4 = p4_ref[...]
    deg = acc4[:, 3:4] + 1.0
    agg = parts_ref[...]
    h = jnp.maximum(
        jnp.dot(x_ref[...], Wu_ref[0], preferred_element_type=_f32)
        + jnp.dot(agg, Wu_ref[1], preferred_element_type=_f32) / deg
        + bu_ref[...], 0.0)
    pos_out = pos_ref[...] + acc4[:, 0:3] / deg
    if decoder:
        prec = (jnp.dot(h, Wo_ref[...], preferred_element_type=_f32)
                + bo_ref[...] + pos_out)
        pr_ref[...] = jnp.concatenate(
            [prec, jnp.zeros((prec.shape[0], D - P), _f32)], axis=1)
    else:
        vis = vis_ref[...]
        z_ref[...] = vis * h + (1.0 - vis) * mt_ref[...]
        posc_ref[...] = vis * pos_out + (1.0 - vis) * pmf_ref[...]


def _upd_call(parts, part4, xn, posn, visn, Wu, bu, mt, pmf, Wo, bo, decoder):
    body = functools.partial(_upd_body, decoder=decoder)
    if decoder:
        out_specs = [pl.BlockSpec((BN, D), lambda i: (i, 0))]
        out_shape = [jax.ShapeDtypeStruct((N, D), _f32)]
    else:
        out_specs = [
            pl.BlockSpec((BN, D), lambda i: (i, 0)),
            pl.BlockSpec((BN, P), lambda i: (i, 0)),
        ]
        out_shape = [
            jax.ShapeDtypeStruct((N, D), _f32),
            jax.ShapeDtypeStruct((N, P), _f32),
        ]

    def wrapped(parts_ref, p4_ref, x_ref, pos_ref, vis_ref, Wu_ref, bu_ref,
                mt_ref, pmf_ref, Wo_ref, bo_ref, *outs):
        if decoder:
            body(parts_ref, p4_ref, x_ref, pos_ref, vis_ref, Wu_ref, bu_ref,
                 mt_ref, pmf_ref, Wo_ref, bo_ref, None, None, outs[0])
        else:
            body(parts_ref, p4_ref, x_ref, pos_ref, vis_ref, Wu_ref, bu_ref,
                 mt_ref, pmf_ref, Wo_ref, bo_ref, outs[0], outs[1], None)

    return pl.pallas_call(
        wrapped,
        grid=(N // BN,),
        in_specs=[
            pl.BlockSpec((BN, D), lambda i: (i, 0)),
            pl.BlockSpec((BN, 4), lambda i: (i, 0)),
            pl.BlockSpec((BN, D), lambda i: (i, 0)),
            pl.BlockSpec((BN, P), lambda i: (i, 0)),
            pl.BlockSpec((BN, 1), lambda i: (i, 0)),
            pl.BlockSpec((2, D, D), lambda i: (0, 0, 0)),
            pl.BlockSpec((1, D), lambda i: (0, 0)),
            pl.BlockSpec((1, D), lambda i: (0, 0)),
            pl.BlockSpec((BN, P), lambda i: (i, 0)),
            pl.BlockSpec((D, P), lambda i: (0, 0)),
            pl.BlockSpec((1, P), lambda i: (0, 0)),
        ],
        out_specs=out_specs,
        out_shape=out_shape,
    )(parts, part4, xn, posn, visn, Wu, bu, mt, pmf, Wo, bo)


@functools.partial(
    pl.kernel,
    out_type=jax.ShapeDtypeStruct((MPAD, D), _f32),
    mesh=_MESH,
    compiler_params=_SC_PARAMS,
    scratch_types=[
        pltpu.VMEM((80,), _i32),
        pltpu.VMEM((80, D), _f32),
        pltpu.SemaphoreType.DMA,
    ],
)
def _sc_mask_gather(table, midx, out, idx_v, buf_v, sem):
    wid = lax.axis_index("s") * 2 + lax.axis_index("c")

    def step(k, _):
        base = wid * MGW + k * 80
        pltpu.sync_copy(midx.at[pl.ds(base, 80)], idx_v)
        pltpu.async_copy(table.at[idx_v], buf_v, sem).wait()
        pltpu.sync_copy(buf_v, out.at[pl.ds(base, 80)])
        return 0

    lax.fori_loop(0, MGW // 80, step, 0)


def kernel(x, pos, edge_index, edge_attr, batch_indices, masked_token,
           enc_W_msg, enc_b_msg, enc_W_upd, enc_b_upd, enc_w_coord,
           dec_W_msg, dec_b_msg, dec_W_upd, dec_b_upd, dec_w_coord,
           dec_W_out, dec_b_out):
    mask_idx, visf, pmf, mask_pad = _consts()
    ones_vis = jnp.ones((N, 1), _f32)
    zerosND = jnp.zeros((N, D), _f32)
    zeros4N = jnp.zeros((N4,), _f32)
    iota4n = jnp.arange(N4, dtype=_i32)
    src = edge_index[0]
    dst = edge_index[1]

    def layer(xn, posn, W_msg, b_msg, w_coord, vis_col):
        Wsd = jnp.stack([W_msg[:D], W_msg[D:2 * D]])
        We3 = jnp.concatenate(
            [W_msg[2 * D:2 * D + DE + 1], jnp.zeros((1, D), _f32)])
        tS, tD = _tables_call(xn, Wsd, b_msg.reshape(1, D))
        posvis = jnp.concatenate(
            [posn[:, 0], posn[:, 1], posn[:, 2], vis_col[:, 0]])
        gS, gD, d2, keepf, rx, ry, rz = _sc_gather(tS, tD, posvis, src, dst)
        eax = jnp.concatenate([edge_attr, d2[:, None], keepf[:, None]],
                              axis=1)
        mk, cf = _edge_call(gS, gD, eax, We3, w_coord)
        parts, part4 = _sc_scatter(mk, dst, cf, keepf, rx, ry, rz, zerosND,
                                   zeros4N, iota4n)
        agg = jnp.concatenate([parts[0, :NSPL], parts[1, :NSPL]], axis=0)
        n4t = jnp.transpose(part4[0].reshape(4, N))
        return agg, n4t

    parts_e, part4_e = layer(x, pos, enc_W_msg, enc_b_msg, enc_w_coord, visf)
    Wu_e = jnp.stack([enc_W_upd[:D], enc_W_upd[D:]])
    dummyWo = jnp.zeros((D, P), _f32)
    dummybo = jnp.zeros((1, P), _f32)
    z, pos_c = _upd_call(parts_e, part4_e, x, pos, visf, Wu_e,
                         enc_b_upd.reshape(1, D), masked_token, pmf,
                         dummyWo, dummybo, decoder=False)

    parts_d, part4_d = layer(z, pos_c, dec_W_msg, dec_b_msg, dec_w_coord,
                             ones_vis)
    Wu_d = jnp.stack([dec_W_upd[:D], dec_W_upd[D:]])
    (prec_pad,) = _upd_call(parts_d, part4_d, z, pos_c, ones_vis, Wu_d,
                            dec_b_upd.reshape(1, D), masked_token, pmf,
                            dec_W_out, dec_b_out.reshape(1, P), decoder=True)

    g = _sc_mask_gather(prec_pad, mask_pad)
    return g[:NUM_MASKED, :P], mask_idx
